# baseline (speedup 1.0000x reference)
import functools

import jax
import jax.numpy as jnp
from jax import lax
from jax.experimental import pallas as pl
from jax.experimental.pallas import tpu as pltpu
from jax.experimental.pallas import tpu_sc as plsc

_LQ = 4096
_LK = 4096
_NCH = 4
_CROWS = _LQ // _NCH
_WIN = 4352
_BR = 256


def _sc_body(h, table_hbm, delta_hbm, out_hbm, table_v, delta_v, win_v, sem):
    cid = lax.axis_index("c")
    sid = lax.axis_index("s")
    wid = sid * 2 + cid
    residue = wid % 8
    m0 = 128 * h + 32 * (wid // 8)

    pltpu.sync_copy(table_hbm, table_v)
    pltpu.sync_copy(delta_hbm, delta_v)

    iot = lax.broadcasted_iota(jnp.int32, (16,), 0)
    c0 = iot + (-120 - residue - 8 * m0) + delta_v[...]

    def build(tb, carry):
        idx = jnp.clip(c0 + tb * 16, 0, 256)
        win_v[pl.ds(pl.multiple_of(tb * 16, 16), 16)] = \
            plsc.load_gather(table_v, [idx])
        return carry

    lax.fori_loop(0, _WIN // 16, build, 0)

    loc_base = residue + 256 * (wid // 8)

    def rows(blk, carry):
        for j in range(8):
            src_off = pl.multiple_of(248 - 64 * blk - 8 * j, 8)
            dst_off = pl.multiple_of((loc_base + 64 * blk + 8 * j) * _LK, 8)
            pltpu.async_copy(
                win_v.at[pl.ds(src_off, _LK)],
                out_hbm.at[pl.ds(dst_off, _LK)], sem)
        return carry

    lax.fori_loop(0, 4, rows, 0)

    def drain(blk, carry):
        for _ in range(8):
            pltpu.make_async_copy(
                win_v.at[pl.ds(0, _LK)], out_hbm.at[pl.ds(0, _LK)], sem
            ).wait()
        return carry

    lax.fori_loop(0, 4, drain, 0)


def _make_sc(h):
    mesh = plsc.VectorSubcoreMesh(core_axis_name="c", subcore_axis_name="s")
    return pl.kernel(
        functools.partial(_sc_body, h),
        out_type=jax.ShapeDtypeStruct((_CROWS * _LK,), jnp.float32),
        mesh=mesh,
        compiler_params=pltpu.CompilerParams(needs_layout_passes=False),
        scratch_types=[
            pltpu.VMEM((272,), jnp.float32),
            pltpu.VMEM((16,), jnp.int32),
            pltpu.VMEM((_WIN,), jnp.float32),
            pltpu.SemaphoreType.DMA,
        ],
    )


_SC_CALLS = [_make_sc(h) for h in range(_NCH)]

_STEPS = _CROWS // _BR


def _retile_first_body(in_ref, out_ref):
    out_ref[...] = in_ref[...].reshape(_BR, _LK)


def _retile_chunk_body(in_ref, carry_ref, out_ref):
    del carry_ref
    out_ref[...] = in_ref[...].reshape(_BR, _LK)


def _retile_first(x3c):
    return pl.pallas_call(
        _retile_first_body,
        grid=(_STEPS,),
        in_specs=[pl.BlockSpec((_BR, _LK // 128, 128), lambda i: (i, 0, 0))],
        out_specs=pl.BlockSpec((_BR, _LK), lambda i: (i, 0)),
        out_shape=jax.ShapeDtypeStruct((_LQ, _LK), jnp.float32),
    )(x3c)


def _make_retile(h):
    def f(x3c, carry):
        return pl.pallas_call(
            _retile_chunk_body,
            grid=(_STEPS,),
            in_specs=[
                pl.BlockSpec((_BR, _LK // 128, 128), lambda i: (i, 0, 0)),
                pl.BlockSpec(memory_space=pl.ANY),
            ],
            out_specs=pl.BlockSpec(
                (_BR, _LK), lambda i, h=h: (i + h * _STEPS, 0)),
            out_shape=jax.ShapeDtypeStruct((_LQ, _LK), jnp.float32),
            input_output_aliases={1: 0},
        )(x3c, carry)
    return f


_RETILE_CALLS = [_retile_first] + [_make_retile(h) for h in range(1, _NCH)]


@jax.jit
def _rel_pos(table_p, delta_arr):
    flats = [sc(table_p, delta_arr) for sc in _SC_CALLS]
    out = _RETILE_CALLS[0](flats[0].reshape(_CROWS, _LK // 128, 128))
    for h in range(1, _NCH):
        out = _RETILE_CALLS[h](
            flats[h].reshape(_CROWS, _LK // 128, 128), out)
    return out


def kernel(embeddings_table, length_q, length_k):
    delta = (length_k - _LK) - (length_q - _LQ)
    table_p = jnp.pad(embeddings_table.astype(jnp.float32), (0, 15))
    delta_arr = jnp.full((16,), delta, dtype=jnp.int32)
    return _rel_pos(table_p, delta_arr)


# speedup vs baseline: 1.7389x; 1.7389x over previous
import functools

import jax
import jax.numpy as jnp
from jax import lax
from jax.experimental import pallas as pl
from jax.experimental.pallas import tpu as pltpu
from jax.experimental.pallas import tpu_sc as plsc

_LQ = 4096
_LK = 4096
_BW = 1024
_WSUB = 1280
_BR = 256


def _sc_body(table_hbm, delta_hbm, band_hbm, table_v, delta_v, win_v, sem):
    cid = lax.axis_index("c")
    sid = lax.axis_index("s")
    wid = sid * 2 + cid
    residue = wid % 8
    m0 = (wid // 8) * 128

    pltpu.sync_copy(table_hbm, table_v)
    pltpu.sync_copy(delta_hbm, delta_v)
    dvec = delta_v[...]
    dsc = dvec[0]
    iot = lax.broadcasted_iota(jnp.int32, (16,), 0)

    cstarts = []
    for q in range(4):
        mq = m0 + 32 * q
        cstart = jnp.clip(
            jnp.right_shift(8 * mq - dsc - 129, 7) * 128, 0, _LK - _BW)
        cstarts.append(cstart)
        c0q = iot + (cstart - 120 - residue - 8 * mq) + dvec

        def build(tb, carry, c0q=c0q, q=q):
            idx = jnp.clip(c0q + tb * 16, 0, 256)
            win_v[pl.ds(pl.multiple_of(q * _WSUB + tb * 16, 8), 16)] = \
                plsc.load_gather(table_v, [idx])
            return carry

        lax.fori_loop(0, _WSUB // 16, build, 0)

    for q in range(4):
        def rows(blk, carry, q=q):
            for j in range(8):
                src_off = pl.multiple_of(
                    q * _WSUB + 248 - 64 * blk - 8 * j, 8)
                dst_off = pl.multiple_of(
                    (residue + 8 * (m0 + 32 * q + 8 * blk + j)) * _BW, 8)
                pltpu.async_copy(
                    win_v.at[pl.ds(src_off, _BW)],
                    band_hbm.at[pl.ds(dst_off, _BW)], sem)
            return carry

        lax.fori_loop(0, 4, rows, 0)

    def drain(blk, carry):
        for _ in range(8):
            pltpu.make_async_copy(
                win_v.at[pl.ds(0, _BW)], band_hbm.at[pl.ds(0, _BW)], sem
            ).wait()
        return carry

    lax.fori_loop(0, 16, drain, 0)


def _sc_call(table_p, delta_arr):
    mesh = plsc.VectorSubcoreMesh(core_axis_name="c", subcore_axis_name="s")
    return pl.kernel(
        _sc_body,
        out_type=jax.ShapeDtypeStruct((_LQ * _BW,), jnp.float32),
        mesh=mesh,
        compiler_params=pltpu.CompilerParams(needs_layout_passes=False),
        scratch_types=[
            pltpu.VMEM((272,), jnp.float32),
            pltpu.VMEM((16,), jnp.int32),
            pltpu.VMEM((4 * _WSUB,), jnp.float32),
            pltpu.SemaphoreType.DMA,
        ],
    )(table_p, delta_arr)


def _tc_body(dsm_ref, cv_ref, band_ref, out_ref):
    i0 = pl.program_id(0) * _BR
    delta = dsm_ref[0]
    c_lo = cv_ref[0]
    c_hi = cv_ref[1]
    cstart = jnp.clip(
        jnp.right_shift(i0 - delta - 129, 7) * 128, 0, _LK - _BW)

    rows_f = i0 + lax.broadcasted_iota(jnp.int32, (_BR, _LK), 0)
    cols_f = lax.broadcasted_iota(jnp.int32, (_BR, _LK), 1)
    d_f = cols_f - rows_f + delta
    out_ref[...] = jnp.where(d_f <= -128, c_lo, c_hi)

    band = band_ref[...].reshape(_BR, _BW)
    rows_w = i0 + lax.broadcasted_iota(jnp.int32, (_BR, _BW), 0)
    cols_w = cstart + lax.broadcasted_iota(jnp.int32, (_BR, _BW), 1)
    d_w = cols_w - rows_w + delta
    mixed = jnp.where(d_w <= -128, c_lo, jnp.where(d_w >= 128, c_hi, band))
    out_ref[:, pl.ds(pl.multiple_of(cstart, 128), _BW)] = mixed


def _tc_call(delta_sm, consts, band3):
    return pl.pallas_call(
        _tc_body,
        grid=(_LQ // _BR,),
        in_specs=[
            pl.BlockSpec(memory_space=pltpu.SMEM),
            pl.BlockSpec(memory_space=pltpu.SMEM),
            pl.BlockSpec((_BR, _BW // 128, 128), lambda i: (i, 0, 0)),
        ],
        out_specs=pl.BlockSpec((_BR, _LK), lambda i: (i, 0)),
        out_shape=jax.ShapeDtypeStruct((_LQ, _LK), jnp.float32),
    )(delta_sm, consts, band3)


@jax.jit
def _rel_pos(table_p, delta_arr, delta_sm, consts):
    band = _sc_call(table_p, delta_arr)
    return _tc_call(delta_sm, consts, band.reshape(_LQ, _BW // 128, 128))


def kernel(embeddings_table, length_q, length_k):
    delta = (length_k - _LK) - (length_q - _LQ)
    table_p = jnp.pad(embeddings_table.astype(jnp.float32), (0, 15))
    delta_arr = jnp.full((16,), delta, dtype=jnp.int32)
    delta_sm = jnp.full((1,), delta, dtype=jnp.int32)
    consts = jnp.stack([embeddings_table[0], embeddings_table[256]])
    return _rel_pos(table_p, delta_arr, delta_sm, consts)


# speedup vs baseline: 1.7496x; 1.0062x over previous
import functools

import jax
import jax.numpy as jnp
from jax import lax
from jax.experimental import pallas as pl
from jax.experimental.pallas import tpu as pltpu
from jax.experimental.pallas import tpu_sc as plsc

_LQ = 4096
_LK = 4096
_BW = 1024
_WSUB = 1280
_BR = 256


def _sc_body(table_hbm, delta_hbm, band_hbm, table_v, delta_v, win_v, sem):
    cid = lax.axis_index("c")
    sid = lax.axis_index("s")
    wid = sid * 2 + cid
    residue = wid % 8
    m0 = (wid // 8) * 128

    pltpu.sync_copy(table_hbm, table_v)
    pltpu.sync_copy(delta_hbm, delta_v)
    dvec = delta_v[...]
    dsc = dvec[0]
    iot = lax.broadcasted_iota(jnp.int32, (16,), 0)

    cstarts = []
    for q in range(4):
        mq = m0 + 32 * q
        cstart = jnp.clip(
            jnp.right_shift(8 * mq - dsc - 129, 7) * 128, 0, _LK - _BW)
        cstarts.append(cstart)
        c0q = iot + (cstart - 120 - residue - 8 * mq) + dvec

        def build(tb, carry, c0q=c0q, q=q):
            idx = jnp.clip(c0q + tb * 16, 0, 256)
            win_v[pl.ds(pl.multiple_of(q * _WSUB + tb * 16, 8), 16)] = \
                plsc.load_gather(table_v, [idx])
            return carry

        lax.fori_loop(0, _WSUB // 16, build, 0)

    for q in range(4):
        def rows(blk, carry, q=q):
            for j in range(8):
                src_off = pl.multiple_of(
                    q * _WSUB + 248 - 64 * blk - 8 * j, 8)
                dst_off = pl.multiple_of(
                    (residue + 8 * (m0 + 32 * q + 8 * blk + j)) * _BW, 8)
                pltpu.async_copy(
                    win_v.at[pl.ds(src_off, _BW)],
                    band_hbm.at[pl.ds(dst_off, _BW)], sem)
            return carry

        lax.fori_loop(0, 4, rows, 0)

    def drain(blk, carry):
        for _ in range(8):
            pltpu.make_async_copy(
                win_v.at[pl.ds(0, _BW)], band_hbm.at[pl.ds(0, _BW)], sem
            ).wait()
        return carry

    lax.fori_loop(0, 16, drain, 0)


def _sc_call(table_p, delta_arr):
    mesh = plsc.VectorSubcoreMesh(core_axis_name="c", subcore_axis_name="s")
    return pl.kernel(
        _sc_body,
        out_type=jax.ShapeDtypeStruct((_LQ * _BW,), jnp.float32),
        mesh=mesh,
        compiler_params=pltpu.CompilerParams(needs_layout_passes=False),
        scratch_types=[
            pltpu.VMEM((272,), jnp.float32),
            pltpu.VMEM((16,), jnp.int32),
            pltpu.VMEM((4 * _WSUB,), jnp.float32),
            pltpu.SemaphoreType.DMA,
        ],
    )(table_p, delta_arr)


def _tc_body(dsm_ref, tsm_ref, band_ref, out_ref):
    i0 = pl.program_id(0) * _BR
    delta = dsm_ref[0]
    c_lo = tsm_ref[0]
    c_hi = tsm_ref[256]
    cstart = jnp.clip(
        jnp.right_shift(i0 - delta - 129, 7) * 128, 0, _LK - _BW)

    cols_f = lax.broadcasted_iota(jnp.int32, (_BR, _LK), 1)
    out_ref[...] = jnp.where(cols_f < cstart + _BW // 2, c_lo, c_hi)

    band = band_ref[...].reshape(_BR, _BW)
    rows_w = i0 + lax.broadcasted_iota(jnp.int32, (_BR, _BW), 0)
    cols_w = cstart + lax.broadcasted_iota(jnp.int32, (_BR, _BW), 1)
    d_w = cols_w - rows_w + delta
    mixed = jnp.where(d_w <= -128, c_lo, jnp.where(d_w >= 128, c_hi, band))
    out_ref[:, pl.ds(pl.multiple_of(cstart, 128), _BW)] = mixed


def _tc_call(delta_arr, table_p, band3):
    return pl.pallas_call(
        _tc_body,
        grid=(_LQ // _BR,),
        in_specs=[
            pl.BlockSpec(memory_space=pltpu.SMEM),
            pl.BlockSpec(memory_space=pltpu.SMEM),
            pl.BlockSpec((_BR, _BW // 128, 128), lambda i: (i, 0, 0)),
        ],
        out_specs=pl.BlockSpec((_BR, _LK), lambda i: (i, 0)),
        out_shape=jax.ShapeDtypeStruct((_LQ, _LK), jnp.float32),
    )(delta_arr, table_p, band3)


@jax.jit
def _rel_pos(table_p, delta_arr):
    band = _sc_call(table_p, delta_arr)
    return _tc_call(delta_arr, table_p, band.reshape(_LQ, _BW // 128, 128))


def kernel(embeddings_table, length_q, length_k):
    delta = (length_k - _LK) - (length_q - _LQ)
    table_p = jnp.pad(embeddings_table.astype(jnp.float32), (0, 15))
    delta_arr = jnp.full((16,), delta, dtype=jnp.int32)
    return _rel_pos(table_p, delta_arr)


# speedup vs baseline: 1.8296x; 1.0457x over previous
import functools

import jax
import jax.numpy as jnp
from jax import lax
from jax.experimental import pallas as pl
from jax.experimental.pallas import tpu as pltpu
from jax.experimental.pallas import tpu_sc as plsc

_LQ = 4096
_LK = 4096
_BW = 1024
_WSUB = 1280
_BR = 512


def _sc_body(table_hbm, delta_hbm, band_hbm, table_v, delta_v, win_v, sem):
    cid = lax.axis_index("c")
    sid = lax.axis_index("s")
    wid = sid * 2 + cid
    residue = wid % 8
    m0 = (wid // 8) * 128

    pltpu.sync_copy(table_hbm, table_v)
    pltpu.sync_copy(delta_hbm, delta_v)
    dvec = delta_v[...]
    dsc = dvec[0]
    iot = lax.broadcasted_iota(jnp.int32, (16,), 0)

    cstarts = []
    for q in range(4):
        mq = m0 + 32 * q
        cstart = jnp.clip(
            jnp.right_shift(8 * mq - dsc - 129, 7) * 128, 0, _LK - _BW)
        cstarts.append(cstart)
        c0q = iot + (cstart - 120 - residue - 8 * mq) + dvec

        def build(tb, carry, c0q=c0q, q=q):
            idx = jnp.clip(c0q + tb * 16, 0, 256)
            win_v[pl.ds(pl.multiple_of(q * _WSUB + tb * 16, 8), 16)] = \
                plsc.load_gather(table_v, [idx])
            return carry

        lax.fori_loop(0, _WSUB // 16, build, 0)

    for q in range(4):
        def rows(blk, carry, q=q):
            for j in range(8):
                src_off = pl.multiple_of(
                    q * _WSUB + 248 - 64 * blk - 8 * j, 8)
                dst_off = pl.multiple_of(
                    (residue + 8 * (m0 + 32 * q + 8 * blk + j)) * _BW, 8)
                pltpu.async_copy(
                    win_v.at[pl.ds(src_off, _BW)],
                    band_hbm.at[pl.ds(dst_off, _BW)], sem)
            return carry

        lax.fori_loop(0, 4, rows, 0)

    def drain(blk, carry):
        for _ in range(8):
            pltpu.make_async_copy(
                win_v.at[pl.ds(0, _BW)], band_hbm.at[pl.ds(0, _BW)], sem
            ).wait()
        return carry

    lax.fori_loop(0, 16, drain, 0)


def _sc_call(table_p, delta_arr):
    mesh = plsc.VectorSubcoreMesh(core_axis_name="c", subcore_axis_name="s")
    return pl.kernel(
        _sc_body,
        out_type=jax.ShapeDtypeStruct((_LQ * _BW,), jnp.float32),
        mesh=mesh,
        compiler_params=pltpu.CompilerParams(needs_layout_passes=False),
        scratch_types=[
            pltpu.VMEM((272,), jnp.float32),
            pltpu.VMEM((16,), jnp.int32),
            pltpu.VMEM((4 * _WSUB,), jnp.float32),
            pltpu.SemaphoreType.DMA,
        ],
    )(table_p, delta_arr)


def _tc_body(dsm_ref, tsm_ref, band_ref, out_ref):
    i0 = pl.program_id(0) * _BR
    delta = dsm_ref[0]
    c_lo = tsm_ref[0]
    c_hi = tsm_ref[256]
    cstart = jnp.clip(
        jnp.right_shift(i0 - delta - 129, 7) * 128, 0, _LK - _BW)

    cols_f = lax.broadcasted_iota(jnp.int32, (_BR, _LK), 1)
    out_ref[...] = jnp.where(cols_f < cstart + _BW // 2, c_lo, c_hi)

    band = band_ref[...].reshape(_BR, _BW)
    rows_w = i0 + lax.broadcasted_iota(jnp.int32, (_BR, _BW), 0)
    cols_w = cstart + lax.broadcasted_iota(jnp.int32, (_BR, _BW), 1)
    d_w = cols_w - rows_w + delta
    mixed = jnp.where(d_w <= -128, c_lo, jnp.where(d_w >= 128, c_hi, band))
    out_ref[:, pl.ds(pl.multiple_of(cstart, 128), _BW)] = mixed


def _tc_call(delta_arr, table_p, band3):
    return pl.pallas_call(
        _tc_body,
        grid=(_LQ // _BR,),
        in_specs=[
            pl.BlockSpec(memory_space=pltpu.SMEM),
            pl.BlockSpec(memory_space=pltpu.SMEM),
            pl.BlockSpec((_BR, _BW // 128, 128), lambda i: (i, 0, 0)),
        ],
        out_specs=pl.BlockSpec((_BR, _LK), lambda i: (i, 0)),
        out_shape=jax.ShapeDtypeStruct((_LQ, _LK), jnp.float32),
    )(delta_arr, table_p, band3)


@jax.jit
def _rel_pos(table_p, delta_arr):
    band = _sc_call(table_p, delta_arr)
    return _tc_call(delta_arr, table_p, band.reshape(_LQ, _BW // 128, 128))


def kernel(embeddings_table, length_q, length_k):
    delta = (length_k - _LK) - (length_q - _LQ)
    table_p = jnp.pad(embeddings_table.astype(jnp.float32), (0, 15))
    delta_arr = jnp.full((16,), delta, dtype=jnp.int32)
    return _rel_pos(table_p, delta_arr)


# speedup vs baseline: 1.8318x; 1.0012x over previous
import functools

import jax
import jax.numpy as jnp
from jax import lax
from jax.experimental import pallas as pl
from jax.experimental.pallas import tpu as pltpu
from jax.experimental.pallas import tpu_sc as plsc

_LQ = 4096
_LK = 4096
_BW = 1024
_WSUB = 1280
_BR = 512


def _sc_body(table_hbm, delta_hbm, band_hbm, table_v, delta_v, win_v, sem):
    cid = lax.axis_index("c")
    sid = lax.axis_index("s")
    wid = sid * 2 + cid
    residue = wid % 8
    m0 = (wid // 8) * 128

    pltpu.sync_copy(table_hbm, table_v)
    pltpu.sync_copy(delta_hbm, delta_v)
    dvec = delta_v[...]
    dsc = dvec[0]
    iot = lax.broadcasted_iota(jnp.int32, (16,), 0)

    for q in range(4):
        mq = m0 + 32 * q
        i0b = 8 * mq - (8 * mq) % _BR
        cstart = jnp.clip(
            jnp.right_shift(i0b - dsc - 129, 7) * 128, 0, _LK - _BW)
        c0q = iot + (cstart - 120 - residue - 8 * mq) + dvec

        def build(tb, carry, c0q=c0q, q=q):
            idx = jnp.clip(c0q + tb * 16, 0, 256)
            win_v[pl.ds(pl.multiple_of(q * _WSUB + tb * 16, 8), 16)] = \
                plsc.load_gather(table_v, [idx])
            return carry

        lax.fori_loop(0, _WSUB // 16, build, 0)

    for q in range(4):
        def rows(blk, carry, q=q):
            for j in range(8):
                src_off = pl.multiple_of(
                    q * _WSUB + 248 - 64 * blk - 8 * j, 8)
                dst_off = pl.multiple_of(
                    (residue + 8 * (m0 + 32 * q + 8 * blk + j)) * _BW, 8)
                pltpu.async_copy(
                    win_v.at[pl.ds(src_off, _BW)],
                    band_hbm.at[pl.ds(dst_off, _BW)], sem)
            return carry

        lax.fori_loop(0, 4, rows, 0)

    def drain(blk, carry):
        for _ in range(8):
            pltpu.make_async_copy(
                win_v.at[pl.ds(0, _BW)], band_hbm.at[pl.ds(0, _BW)], sem
            ).wait()
        return carry

    lax.fori_loop(0, 16, drain, 0)


def _sc_call(table_p, delta_arr):
    mesh = plsc.VectorSubcoreMesh(core_axis_name="c", subcore_axis_name="s")
    return pl.kernel(
        _sc_body,
        out_type=jax.ShapeDtypeStruct((_LQ * _BW,), jnp.float32),
        mesh=mesh,
        compiler_params=pltpu.CompilerParams(needs_layout_passes=False),
        scratch_types=[
            pltpu.VMEM((272,), jnp.float32),
            pltpu.VMEM((16,), jnp.int32),
            pltpu.VMEM((4 * _WSUB,), jnp.float32),
            pltpu.SemaphoreType.DMA,
        ],
    )(table_p, delta_arr)


def _tc_body(dsm_ref, tsm_ref, band_ref, out_ref):
    i0 = pl.program_id(0) * _BR
    delta = dsm_ref[0]
    c_lo = tsm_ref[0]
    c_hi = tsm_ref[256]
    cstart = jnp.clip(
        jnp.right_shift(i0 - delta - 129, 7) * 128, 0, _LK - _BW)

    cols_f = lax.broadcasted_iota(jnp.int32, (_BR, _LK), 1)
    out_ref[...] = jnp.where(cols_f < cstart + _BW // 2, c_lo, c_hi)

    band = band_ref[...].reshape(_BR, _BW)
    rows_w = i0 + lax.broadcasted_iota(jnp.int32, (_BR, _BW), 0)
    cols_w = cstart + lax.broadcasted_iota(jnp.int32, (_BR, _BW), 1)
    d_w = cols_w - rows_w + delta
    mixed = jnp.where(d_w <= -128, c_lo, jnp.where(d_w >= 128, c_hi, band))
    out_ref[:, pl.ds(pl.multiple_of(cstart, 128), _BW)] = mixed


def _tc_call(delta_arr, table_p, band3):
    return pl.pallas_call(
        _tc_body,
        grid=(_LQ // _BR,),
        in_specs=[
            pl.BlockSpec(memory_space=pltpu.SMEM),
            pl.BlockSpec(memory_space=pltpu.SMEM),
            pl.BlockSpec((_BR, _BW // 128, 128), lambda i: (i, 0, 0)),
        ],
        out_specs=pl.BlockSpec((_BR, _LK), lambda i: (i, 0)),
        out_shape=jax.ShapeDtypeStruct((_LQ, _LK), jnp.float32),
    )(delta_arr, table_p, band3)


@jax.jit
def _rel_pos(table_p, delta_arr):
    band = _sc_call(table_p, delta_arr)
    return _tc_call(delta_arr, table_p, band.reshape(_LQ, _BW // 128, 128))


def kernel(embeddings_table, length_q, length_k):
    delta = (length_k - _LK) - (length_q - _LQ)
    table_p = jnp.pad(embeddings_table.astype(jnp.float32), (0, 15))
    delta_arr = jnp.full((16,), delta, dtype=jnp.int32)
    return _rel_pos(table_p, delta_arr)
